# Initial kernel scaffold; baseline (speedup 1.0000x reference)
import jax
import jax.numpy as jnp
from jax.experimental import pallas as pl


def kernel(premise, hypothesis, word_emb):
    raise NotImplementedError("write your pallas kernel here")


# speedup vs baseline: 1.2179x; 1.2179x over previous
import functools

import jax
import jax.numpy as jnp
from jax import lax
from jax.experimental import pallas as pl
from jax.experimental.pallas import tpu as pltpu, tpu_sc as plsc

VOCAB = 1000000
DIM = 64
BATCH = 4096
SEQ = 50

_INFO = plsc.get_sparse_core_info()
NC, NS = _INFO.num_cores, _INFO.num_subcores
NW = NC * NS
TOTAL = BATCH * SEQ
PER_W = TOTAL // NW
CHUNK = 128
NCHUNK = PER_W // CHUNK


def _make_gather():
    mesh = plsc.VectorSubcoreMesh(core_axis_name="c", subcore_axis_name="s")

    @functools.partial(
        pl.kernel,
        mesh=mesh,
        out_type=[
            jax.ShapeDtypeStruct((TOTAL, DIM), jnp.float32),
            jax.ShapeDtypeStruct((TOTAL, DIM), jnp.float32),
        ],
        scratch_types=[
            pltpu.VMEM((NCHUNK, CHUNK), jnp.int32),
            pltpu.VMEM((CHUNK, DIM), jnp.float32),
            pltpu.SemaphoreType.DMA,
        ],
        compiler_params=pltpu.CompilerParams(use_tc_tiling_on_sc=False),
    )
    def k(table_hbm, pidx_hbm, hidx_hbm, p_out, h_out, idx_v, rows_v, sem):
        wid = lax.axis_index("s") * NC + lax.axis_index("c")
        base = wid * PER_W
        for idx_hbm, out_hbm in ((pidx_hbm, p_out), (hidx_hbm, h_out)):
            pltpu.sync_copy(idx_hbm.at[wid], idx_v)

            def chunk_body(j, carry, out_hbm=out_hbm):
                pltpu.async_copy(table_hbm.at[idx_v.at[j]], rows_v, sem).wait()
                pltpu.sync_copy(rows_v, out_hbm.at[pl.ds(base + j * CHUNK, CHUNK)])
                return carry

            lax.fori_loop(0, NCHUNK, chunk_body, 0)

    return k


_gather = _make_gather()


def kernel(premise, hypothesis, word_emb):
    pidx = premise.reshape(NW, NCHUNK, CHUNK)
    hidx = hypothesis.reshape(NW, NCHUNK, CHUNK)
    p_rows, h_rows = _gather(word_emb, pidx, hidx)
    return (
        p_rows.reshape(BATCH, SEQ, DIM),
        h_rows.reshape(BATCH, SEQ, DIM),
    )


# speedup vs baseline: 1.3108x; 1.0764x over previous
import functools

import jax
import jax.numpy as jnp
from jax import lax
from jax.experimental import pallas as pl
from jax.experimental.pallas import tpu as pltpu, tpu_sc as plsc

VOCAB = 1000000
DIM = 64
BATCH = 4096
SEQ = 50

_INFO = plsc.get_sparse_core_info()
NC, NS = _INFO.num_cores, _INFO.num_subcores
NW = NC * NS
TOTAL = BATCH * SEQ
PER_W = TOTAL // NW
CHUNK = 128
NCHUNK = PER_W // CHUNK
NBUF = 10


def _make_gather():
    mesh = plsc.VectorSubcoreMesh(core_axis_name="c", subcore_axis_name="s")

    @functools.partial(
        pl.kernel,
        mesh=mesh,
        out_type=[
            jax.ShapeDtypeStruct((TOTAL, DIM), jnp.float32),
            jax.ShapeDtypeStruct((TOTAL, DIM), jnp.float32),
        ],
        scratch_types=[
            pltpu.VMEM((NCHUNK, CHUNK), jnp.int32),
            pltpu.VMEM((NBUF, CHUNK, DIM), jnp.float32),
            pltpu.SemaphoreType.DMA,
        ],
        compiler_params=pltpu.CompilerParams(use_tc_tiling_on_sc=False),
    )
    def k(table_hbm, pidx_hbm, hidx_hbm, p_out, h_out, idx_v, rows_v, sem):
        wid = lax.axis_index("s") * NC + lax.axis_index("c")
        base = wid * PER_W
        for idx_hbm, out_hbm in ((pidx_hbm, p_out), (hidx_hbm, h_out)):
            pltpu.sync_copy(idx_hbm.at[wid], idx_v)

            def prime(b, carry):
                pltpu.async_copy(table_hbm.at[idx_v.at[b]], rows_v.at[b], sem)
                return carry

            lax.fori_loop(0, NBUF, prime, 0)

            def chunk_body(j, carry, out_hbm=out_hbm):
                b = lax.rem(j, NBUF)
                pltpu.make_async_copy(
                    table_hbm.at[idx_v.at[0]], rows_v.at[0], sem
                ).wait()
                pltpu.sync_copy(
                    rows_v.at[b], out_hbm.at[pl.ds(base + j * CHUNK, CHUNK)]
                )

                @pl.when(j + NBUF < NCHUNK)
                def _():
                    pltpu.async_copy(
                        table_hbm.at[idx_v.at[j + NBUF]], rows_v.at[b], sem
                    )

                return carry

            lax.fori_loop(0, NCHUNK, chunk_body, 0)

    return k


_gather = _make_gather()


def kernel(premise, hypothesis, word_emb):
    pidx = premise.reshape(NW, NCHUNK, CHUNK)
    hidx = hypothesis.reshape(NW, NCHUNK, CHUNK)
    p_rows, h_rows = _gather(word_emb, pidx, hidx)
    return (
        p_rows.reshape(BATCH, SEQ, DIM),
        h_rows.reshape(BATCH, SEQ, DIM),
    )
